# baseline (speedup 1.0000x reference)
import jax
import jax.numpy as jnp
from jax import lax
from jax.experimental import pallas as pl
from jax.experimental.pallas import tpu as pltpu
from jax.experimental.pallas import tpu_sc as plsc

_VOCAB = 30528
_D = 768
_S = 2048
_B = 4
_EPS = 1e-12

_NC = 2
_NS = 16
_NW = _NC * _NS
_NTOK = _B * _S
_TPW = _NTOK // _NW
_C = 16
_NCHUNK = _TPW // _C
_NV = _D // 16


def _rsqrt_vec(v):
    bits = lax.bitcast_convert_type(v, jnp.int32)
    y = lax.bitcast_convert_type(jnp.int32(0x5F3759DF) - (bits >> 1), jnp.float32)
    half = v * 0.5
    for _ in range(2):
        y = y * (1.5 - half * y * y)
    return y


def _sc_body(ids_hbm, tids_hbm, word_hbm, pos_hbm, type_hbm, gamma_hbm,
             beta_hbm, out_hbm,
             idx0, idx1, tid0, tid1,
             wbuf0, wbuf1, pbuf0, pbuf1, xbuf, obuf0, obuf1,
             t01buf, gbuf, bbuf, mbuf, ibuf,
             wsem0, wsem1, psem0, psem1, osem0, osem1):
    idx = [idx0, idx1]
    tid = [tid0, tid1]
    wbuf = [wbuf0, wbuf1]
    pbuf = [pbuf0, pbuf1]
    obuf = [obuf0, obuf1]
    wsem = [wsem0, wsem1]
    psem = [psem0, psem1]
    osem = [osem0, osem1]

    cid = lax.axis_index("c")
    sid = lax.axis_index("s")
    wid = sid * _NC + cid
    tok0 = wid * _TPW
    s0 = lax.rem(tok0, _S)

    pltpu.sync_copy(gamma_hbm, gbuf)
    pltpu.sync_copy(beta_hbm, bbuf)
    pltpu.sync_copy(type_hbm, t01buf)


    def fire(kk, b2):
        base = tok0 + kk * _C
        sb = s0 + kk * _C
        pltpu.sync_copy(ids_hbm.at[pl.ds(base, _C)], idx[b2])
        pltpu.sync_copy(tids_hbm.at[pl.ds(base, _C)], tid[b2])
        pltpu.async_copy(word_hbm.at[idx[b2]], wbuf[b2], wsem[b2])
        pltpu.async_copy(pos_hbm.at[pl.ds(sb, _C)], pbuf[b2], psem[b2])

    fire(0, 0)
    fire(1, 1)

    def chunk(kk, b2):
        base = tok0 + kk * _C
        pltpu.make_async_copy(word_hbm.at[idx[b2]], wbuf[b2],
                              wsem[b2]).wait()
        pltpu.make_async_copy(pos_hbm.at[pl.ds(s0, _C)], pbuf[b2],
                              psem[b2]).wait()
        tidvec = tid[b2][...]

        wb = wbuf[b2]
        pb = pbuf[b2]
        ob = obuf[b2]

        for h in range(0, _C, 8):
            preds = []
            for i in range(8):
                tb = lax.gather(
                    tidvec, jnp.full((16, 1), h + i, jnp.int32),
                    lax.GatherDimensionNumbers(
                        offset_dims=(), collapsed_slice_dims=(0,),
                        start_index_map=(0,)),
                    (1,), mode=lax.GatherScatterMode.PROMISE_IN_BOUNDS)
                preds.append(tb > 0)
            zero = jnp.zeros((16,), jnp.float32)
            carry0 = (tuple(zero for _ in range(8)),
                      tuple(zero for _ in range(8)))

            @plsc.parallel_loop(0, _NV, carry=carry0)
            def _pass_a(j, carry):
                accs, acc2s = carry
                sl = pl.ds(j * 16, 16)
                t0 = t01buf[0, sl]
                t1 = t01buf[1, sl]
                na, n2 = [], []
                for i in range(8):
                    x = wb[h + i, sl] + (pb[h + i, sl]
                                         + jnp.where(preds[i], t1, t0))
                    xbuf[h + i, sl] = x
                    na.append(accs[i] + x)
                    n2.append(acc2s[i] + x * x)
                return (tuple(na), tuple(n2))

            accs, acc2s = _pass_a
            for i in range(8):
                tot = jnp.sum(accs[i])
                tot2 = jnp.sum(acc2s[i])
                mean = tot * (1.0 / _D)
                var = tot2 * (1.0 / _D) - mean * mean
                inv = _rsqrt_vec(jnp.full((16,), var + _EPS, jnp.float32))
                mbuf[h + i, :] = jnp.full((16,), mean, jnp.float32)
                ibuf[h + i, :] = inv

        @pl.when(kk >= 2)
        def _():
            pltpu.make_async_copy(ob, out_hbm.at[pl.ds(base, _C)],
                                  osem[b2]).wait()

        for h in range(0, _C, 8):
            mb = [mbuf[i, :] for i in range(h, h + 8)]
            iv = [ibuf[i, :] for i in range(h, h + 8)]

            @plsc.parallel_loop(0, _NV)
            def _pass_b(j):
                sl = pl.ds(j * 16, 16)
                g = gbuf[sl]
                bt = bbuf[sl]
                for i in range(8):
                    ob[h + i, sl] = (xbuf[h + i, sl] - mb[i]) * iv[i] * g + bt

        pltpu.async_copy(ob, out_hbm.at[pl.ds(base, _C)], osem[b2])

        @pl.when(kk + 2 < _NCHUNK)
        def _():
            fire(kk + 2, b2)

    @pl.loop(0, _NCHUNK, step=2, unroll=2)
    def _chunk_pair(r):
        chunk(r, 0)
        chunk(r + 1, 1)

    for last in (_NCHUNK - 2, _NCHUNK - 1):
        b2 = last % 2
        base = tok0 + last * _C
        pltpu.make_async_copy(obuf[b2], out_hbm.at[pl.ds(base, _C)],
                              osem[b2]).wait()


@jax.jit
def _run(ids, tids, word_emb, pos_emb, type_emb, gamma, beta):
    mesh = plsc.VectorSubcoreMesh(
        core_axis_name="c", subcore_axis_name="s", num_cores=_NC,
        num_subcores=_NS)
    f = pl.kernel(
        _sc_body,
        out_type=jax.ShapeDtypeStruct((_NTOK, _D), jnp.float32),
        mesh=mesh,
        compiler_params=pltpu.CompilerParams(needs_layout_passes=False),
        scratch_types=[
            pltpu.VMEM((_C,), jnp.int32),
            pltpu.VMEM((_C,), jnp.int32),
            pltpu.VMEM((_C,), jnp.int32),
            pltpu.VMEM((_C,), jnp.int32),
            pltpu.VMEM((_C, _D), jnp.float32),
            pltpu.VMEM((_C, _D), jnp.float32),
            pltpu.VMEM((_C, _D), jnp.float32),
            pltpu.VMEM((_C, _D), jnp.float32),
            pltpu.VMEM((_C, _D), jnp.float32),
            pltpu.VMEM((_C, _D), jnp.float32),
            pltpu.VMEM((_C, _D), jnp.float32),
            pltpu.VMEM((2, _D), jnp.float32),
            pltpu.VMEM((_D,), jnp.float32),
            pltpu.VMEM((_D,), jnp.float32),
            pltpu.VMEM((_C, 16), jnp.float32),
            pltpu.VMEM((_C, 16), jnp.float32),
        ] + [pltpu.SemaphoreType.DMA] * 6,
    )
    return f(ids, tids, word_emb, pos_emb, type_emb, gamma, beta)


def kernel(input_ids, token_type_ids, word_emb, pos_emb, type_emb, gamma,
           beta):
    ids = input_ids.reshape(_NTOK).astype(jnp.int32)
    tids = token_type_ids.reshape(_NTOK).astype(jnp.int32)
    out = _run(ids, tids, word_emb, pos_emb, type_emb, gamma, beta)
    return out.reshape(_B, _S, _D)


# speedup vs baseline: 1.0469x; 1.0469x over previous
import jax
import jax.numpy as jnp
from jax import lax
from jax.experimental import pallas as pl
from jax.experimental.pallas import tpu as pltpu
from jax.experimental.pallas import tpu_sc as plsc

_VOCAB = 30528
_D = 768
_S = 2048
_B = 4
_EPS = 1e-12

_NC = 2
_NS = 16
_NW = _NC * _NS
_NTOK = _B * _S
_TPW = _NTOK // _NW
_C = 16
_NCHUNK = _TPW // _C
_NV = _D // 16


def _rsqrt_vec(v):
    bits = lax.bitcast_convert_type(v, jnp.int32)
    y = lax.bitcast_convert_type(jnp.int32(0x5F3759DF) - (bits >> 1), jnp.float32)
    half = v * 0.5
    for _ in range(2):
        y = y * (1.5 - half * y * y)
    return y


def _sc_body(ids_hbm, tids_hbm, word_hbm, pos_hbm, type_hbm, gamma_hbm,
             beta_hbm, out_hbm,
             idx0, idx1, tid0, tid1,
             wbuf0, wbuf1, pbuf0, pbuf1, xbuf, obuf0, obuf1,
             t01buf, gbuf, bbuf, mbuf, ibuf,
             wsem0, wsem1, psem0, psem1, osem0, osem1):
    idx = [idx0, idx1]
    tid = [tid0, tid1]
    wbuf = [wbuf0, wbuf1]
    pbuf = [pbuf0, pbuf1]
    obuf = [obuf0, obuf1]
    wsem = [wsem0, wsem1]
    psem = [psem0, psem1]
    osem = [osem0, osem1]

    cid = lax.axis_index("c")
    sid = lax.axis_index("s")
    wid = sid * _NC + cid
    tok0 = wid * _TPW
    s0 = lax.rem(tok0, _S)

    pltpu.sync_copy(gamma_hbm, gbuf)
    pltpu.sync_copy(beta_hbm, bbuf)
    pltpu.sync_copy(type_hbm, t01buf)


    def fire(kk, b2):
        base = tok0 + kk * _C
        sb = s0 + kk * _C
        pltpu.sync_copy(ids_hbm.at[pl.ds(base, _C)], idx[b2])
        pltpu.sync_copy(tids_hbm.at[pl.ds(base, _C)], tid[b2])
        pltpu.async_copy(word_hbm.at[idx[b2]], wbuf[b2], wsem[b2])
        pltpu.async_copy(pos_hbm.at[pl.ds(sb, _C)], pbuf[b2], psem[b2])

    fire(0, 0)
    fire(1, 1)

    def chunk(kk, b2):
        base = tok0 + kk * _C
        pltpu.make_async_copy(word_hbm.at[idx[b2]], wbuf[b2],
                              wsem[b2]).wait()
        pltpu.make_async_copy(pos_hbm.at[pl.ds(s0, _C)], pbuf[b2],
                              psem[b2]).wait()
        tidvec = tid[b2][...]

        wb = wbuf[b2]
        pb = pbuf[b2]
        ob = obuf[b2]

        for h in range(0, _C, 8):
            preds = []
            for i in range(8):
                tb = lax.gather(
                    tidvec, jnp.full((16, 1), h + i, jnp.int32),
                    lax.GatherDimensionNumbers(
                        offset_dims=(), collapsed_slice_dims=(0,),
                        start_index_map=(0,)),
                    (1,), mode=lax.GatherScatterMode.PROMISE_IN_BOUNDS)
                preds.append(tb > 0)
            zero = jnp.zeros((16,), jnp.float32)
            carry0 = (tuple(zero for _ in range(8)),
                      tuple(zero for _ in range(8)))

            @plsc.parallel_loop(0, _NV, carry=carry0)
            def _pass_a(j, carry):
                accs, acc2s = carry
                sl = pl.ds(j * 16, 16)
                t0 = t01buf[0, sl]
                t1 = t01buf[1, sl]
                na, n2 = [], []
                for i in range(8):
                    x = wb[h + i, sl] + (pb[h + i, sl]
                                         + jnp.where(preds[i], t1, t0))
                    xbuf[h + i, sl] = x
                    na.append(accs[i] + x)
                    n2.append(acc2s[i] + x * x)
                return (tuple(na), tuple(n2))

            accs, acc2s = _pass_a
            for i in range(8):
                tot = jnp.sum(accs[i])
                tot2 = jnp.sum(acc2s[i])
                mean = tot * (1.0 / _D)
                var = tot2 * (1.0 / _D) - mean * mean
                inv = _rsqrt_vec(jnp.full((16,), var + _EPS, jnp.float32))
                mbuf[h + i, :] = jnp.full((16,), mean, jnp.float32)
                ibuf[h + i, :] = inv

        @pl.when(kk + 2 < _NCHUNK)
        def _():
            fire(kk + 2, b2)

        @pl.when(kk >= 2)
        def _():
            pltpu.make_async_copy(ob, out_hbm.at[pl.ds(base, _C)],
                                  osem[b2]).wait()

        for h in range(0, _C, 8):
            mb = [mbuf[i, :] for i in range(h, h + 8)]
            iv = [ibuf[i, :] for i in range(h, h + 8)]

            @plsc.parallel_loop(0, _NV)
            def _pass_b(j):
                sl = pl.ds(j * 16, 16)
                g = gbuf[sl]
                bt = bbuf[sl]
                for i in range(8):
                    ob[h + i, sl] = (xbuf[h + i, sl] - mb[i]) * iv[i] * g + bt

        pltpu.async_copy(ob, out_hbm.at[pl.ds(base, _C)], osem[b2])

    @pl.loop(0, _NCHUNK, step=2)
    def _chunk_pair(r):
        chunk(r, 0)
        chunk(r + 1, 1)

    for last in (_NCHUNK - 2, _NCHUNK - 1):
        b2 = last % 2
        base = tok0 + last * _C
        pltpu.make_async_copy(obuf[b2], out_hbm.at[pl.ds(base, _C)],
                              osem[b2]).wait()


@jax.jit
def _run(ids, tids, word_emb, pos_emb, type_emb, gamma, beta):
    mesh = plsc.VectorSubcoreMesh(
        core_axis_name="c", subcore_axis_name="s", num_cores=_NC,
        num_subcores=_NS)
    f = pl.kernel(
        _sc_body,
        out_type=jax.ShapeDtypeStruct((_NTOK, _D), jnp.float32),
        mesh=mesh,
        compiler_params=pltpu.CompilerParams(needs_layout_passes=False),
        scratch_types=[
            pltpu.VMEM((_C,), jnp.int32),
            pltpu.VMEM((_C,), jnp.int32),
            pltpu.VMEM((_C,), jnp.int32),
            pltpu.VMEM((_C,), jnp.int32),
            pltpu.VMEM((_C, _D), jnp.float32),
            pltpu.VMEM((_C, _D), jnp.float32),
            pltpu.VMEM((_C, _D), jnp.float32),
            pltpu.VMEM((_C, _D), jnp.float32),
            pltpu.VMEM((_C, _D), jnp.float32),
            pltpu.VMEM((_C, _D), jnp.float32),
            pltpu.VMEM((_C, _D), jnp.float32),
            pltpu.VMEM((2, _D), jnp.float32),
            pltpu.VMEM((_D,), jnp.float32),
            pltpu.VMEM((_D,), jnp.float32),
            pltpu.VMEM((_C, 16), jnp.float32),
            pltpu.VMEM((_C, 16), jnp.float32),
        ] + [pltpu.SemaphoreType.DMA] * 6,
    )
    return f(ids, tids, word_emb, pos_emb, type_emb, gamma, beta)


def kernel(input_ids, token_type_ids, word_emb, pos_emb, type_emb, gamma,
           beta):
    ids = input_ids.reshape(_NTOK).astype(jnp.int32)
    tids = token_type_ids.reshape(_NTOK).astype(jnp.int32)
    out = _run(ids, tids, word_emb, pos_emb, type_emb, gamma, beta)
    return out.reshape(_B, _S, _D)
